# baseline (speedup 1.0000x reference)
import functools

import jax
import jax.numpy as jnp
from jax import lax
from jax.experimental import pallas as pl
from jax.experimental.pallas import tpu as pltpu
from jax.experimental.pallas import tpu_sc as plsc

N = 10000
E = 320000
D = 128
H = 128
OUT = 2
NB = 160000
NA = 150000
NPR = 100000
NI = 50000

NC = 2
NS = 16
L = 16

N_PAD = 10240
R_TC = 512

E_PER_SC = E // NC
E_PER_TILE = E_PER_SC // NS
SEG_CH = 80
SEG_ITERS = E_PER_TILE // SEG_CH
N_SH_PER_TILE = N_PAD // NS

NB_P = 160256
NA_P = 150016
NPR_P = 100352
NI_P = 50176
POOL_MAX_W = 5008

_mesh = plsc.VectorSubcoreMesh(core_axis_name="c", subcore_axis_name="s")


@functools.partial(
    pl.kernel,
    out_type=jax.ShapeDtypeStruct((NC, N_PAD, D), jnp.float32),
    mesh=_mesh,
    scratch_types=[
        pltpu.VMEM((E_PER_TILE,), jnp.int32),
        pltpu.VMEM((E_PER_TILE,), jnp.int32),
        pltpu.VMEM((SEG_CH, D), jnp.float32),
        pltpu.VMEM((SEG_CH, D), jnp.float32),
        pltpu.VMEM_SHARED((N_PAD, D), jnp.float32),
        pltpu.SemaphoreType.DMA,
        pltpu.SemaphoreType.DMA,
    ],
    compiler_params=pltpu.CompilerParams(needs_layout_passes=False),
)
def _segment_sum_sc(x_hbm, src_hbm, dst_hbm, zeros_hbm, out_hbm, srcs_v,
                    dsts_v, rows_a, rows_b, agg_sh, sem_a, sem_b):
    c = lax.axis_index("c")
    s = lax.axis_index("s")

    pltpu.sync_copy(zeros_hbm, agg_sh.at[pl.ds(s * N_SH_PER_TILE,
                                               N_SH_PER_TILE)])

    tile_base = c * E_PER_SC + s * E_PER_TILE
    pltpu.sync_copy(src_hbm.at[pl.ds(tile_base, E_PER_TILE)], srcs_v)
    pltpu.sync_copy(dst_hbm.at[pl.ds(tile_base, E_PER_TILE)], dsts_v)
    plsc.subcore_barrier()

    def _gather(off, rows, sem):
        return pltpu.async_copy(
            x_hbm.at[srcs_v.at[pl.ds(off, SEG_CH)]], rows, sem)

    def _wait(off, rows, sem):
        pltpu.make_async_copy(
            x_hbm.at[srcs_v.at[pl.ds(off, SEG_CH)]], rows, sem).wait()

    def _scatter_add(off, rows):
        pltpu.sync_copy(rows, agg_sh.at[dsts_v.at[pl.ds(off, SEG_CH)]],
                        add=True)

    _gather(0, rows_a, sem_a)

    def _edge_body(k, _):
        offa = 2 * k * SEG_CH
        offb = offa + SEG_CH
        offn = offa + 2 * SEG_CH
        _gather(offb, rows_b, sem_b)
        _wait(offa, rows_a, sem_a)
        _scatter_add(offa, rows_a)
        _gather(offn, rows_a, sem_a)
        _wait(offb, rows_b, sem_b)
        _scatter_add(offb, rows_b)
        return 0

    lax.fori_loop(0, (SEG_ITERS - 1) // 2, _edge_body, 0)
    tail = (SEG_ITERS - 1) * SEG_CH
    _wait(tail, rows_a, sem_a)
    _scatter_add(tail, rows_a)
    plsc.subcore_barrier()

    pltpu.sync_copy(agg_sh.at[pl.ds(s * N_SH_PER_TILE, N_SH_PER_TILE)],
                    out_hbm.at[c, pl.ds(s * N_SH_PER_TILE, N_SH_PER_TILE)])


def _encoder_body(x_ref, agg_ref, wmsg_ref, wself_ref, bg_ref, wcat_ref,
                  bcat_ref, t_ref):
    agg = agg_ref[0] + agg_ref[1]
    pre = (
        lax.dot_general(agg, wmsg_ref[...], (((1,), (0,)), ((), ())),
                        precision=lax.Precision.HIGHEST)
        + lax.dot_general(x_ref[...], wself_ref[...], (((1,), (0,)), ((), ())),
                          precision=lax.Precision.HIGHEST)
        + bg_ref[...]
    )
    nr = jnp.maximum(pre, 0.0)
    t = lax.dot_general(wcat_ref[...], nr, (((0,), (1,)), ((), ())),
                        precision=lax.Precision.HIGHEST)
    t_ref[...] = t + bcat_ref[...]


_encoder_tc = pl.pallas_call(
    _encoder_body,
    grid=(N_PAD // R_TC,),
    in_specs=[
        pl.BlockSpec((R_TC, D), lambda i: (i, 0)),
        pl.BlockSpec((NC, R_TC, D), lambda i: (0, i, 0)),
        pl.BlockSpec((D, H), lambda i: (0, 0)),
        pl.BlockSpec((D, H), lambda i: (0, 0)),
        pl.BlockSpec((1, H), lambda i: (0, 0)),
        pl.BlockSpec((H, 16), lambda i: (0, 0)),
        pl.BlockSpec((16, 1), lambda i: (0, 0)),
    ],
    out_specs=pl.BlockSpec((16, R_TC), lambda i: (0, i)),
    out_shape=jax.ShapeDtypeStruct((16, N_PAD), jnp.float32),
)

_POOL_PHASES = (
    (NB_P, 2, (0, 1, 0, 1), (0, 0)),
    (NA_P, 3, (2, 3, 4, 5), (0, 2, 0)),
    (NPR_P, 4, (6, 7, 8, 9), (0, 2, 2, 0)),
    (NI_P, 4, (10, 11, 12, 13), (0, 2, 2, 2)),
)


@functools.partial(
    pl.kernel,
    out_type=tuple(
        jax.ShapeDtypeStruct((OUT * cnt,), jnp.float32)
        for (cnt, _, _, _) in _POOL_PHASES
    ),
    mesh=_mesh,
    scratch_types=[
        pltpu.VMEM((4 * N,), jnp.float32),
        pltpu.VMEM((4 * POOL_MAX_W,), jnp.int32),
        pltpu.VMEM((OUT * POOL_MAX_W,), jnp.float32),
    ],
    compiler_params=pltpu.CompilerParams(needs_layout_passes=False),
)
def _pools_sc(t_hbm, bidx_hbm, aidx_hbm, pidx_hbm, iidx_hbm,
              bonds_hbm, angles_hbm, propers_hbm, impropers_hbm,
              tab_v, idx_v, out_v):
    c = lax.axis_index("c")
    s = lax.axis_index("s")
    wid = s * NC + c
    lanes = lax.iota(jnp.int32, L)

    idx_refs = (bidx_hbm, aidx_hbm, pidx_hbm, iidx_hbm)
    out_refs = (bonds_hbm, angles_hbm, propers_hbm, impropers_hbm)

    for phase, (cnt, k, trows, tbase) in enumerate(_POOL_PHASES):
        w = cnt // (NC * NS)
        slab = k * w
        base = wid * w
        for j, r in enumerate(trows):
            pltpu.sync_copy(t_hbm.at[pl.ds(r * N_PAD, N)],
                            tab_v.at[pl.ds(j * N, N)])
        pltpu.sync_copy(idx_refs[phase].at[pl.ds(wid * slab, slab)],
                        idx_v.at[pl.ds(0, slab)])
        posk = lanes * k
        pos2 = lanes * OUT

        def _group_body(g, _, k=k, tbase=tbase, posk=posk, pos2=pos2):
            offk = g * (L * k)
            off2 = g * (L * OUT)
            ivecs = [plsc.load_gather(idx_v, [posk + (offk + j)])
                     for j in range(k)]
            for col in range(OUT):
                acc = plsc.load_gather(
                    tab_v, [ivecs[0] + (tbase[0] + col) * N])
                for j in range(1, k):
                    acc = acc + plsc.load_gather(
                        tab_v, [ivecs[j] + (tbase[j] + col) * N])
                plsc.store_scatter(out_v, [pos2 + (off2 + col)], acc)
            return 0

        lax.fori_loop(0, w // L, _group_body, 0)
        pltpu.sync_copy(out_v.at[pl.ds(0, OUT * w)],
                        out_refs[phase].at[pl.ds(OUT * base, OUT * w)])


def _pad_rows(a, rows):
    return jnp.pad(a, ((0, rows - a.shape[0]), (0, 0)))


def kernel(x, edge_index, bond_idx, angle_idx, proper_idx, improper_idx,
           W_msg, W_self, b_gnn, W_atoms, b_atoms, W_bonds, b_bonds,
           W_angles, b_angles, W_propers, b_propers, W_impropers, b_impropers):
    f32 = jnp.float32

    wb = W_bonds[:H] + W_bonds[H:]
    wa02 = W_angles[:H] + W_angles[2 * H:]
    wa1 = 2.0 * W_angles[H:2 * H]
    wp03 = W_propers[:H] + W_propers[3 * H:]
    wp12 = W_propers[H:2 * H] + W_propers[2 * H:3 * H]
    wi0 = 3.0 * W_impropers[:H]
    wi123 = (W_impropers[H:2 * H] + W_impropers[2 * H:3 * H]
             + W_impropers[3 * H:])
    wcat = jnp.concatenate(
        [wb, wa02, wa1, wp03, wp12, wi0, wi123, W_atoms], axis=1)
    z2 = jnp.zeros((OUT,), f32)
    bcat = jnp.concatenate(
        [b_bonds, z2, 2.0 * b_angles, b_propers, z2, 3.0 * b_impropers, z2,
         b_atoms]).reshape(16, 1)

    zeros_blk = jnp.zeros((N_SH_PER_TILE, D), f32)
    agg_pair = _segment_sum_sc(x, edge_index[0], edge_index[1], zeros_blk)

    x_pad = _pad_rows(x, N_PAD)
    t = _encoder_tc(x_pad, agg_pair, W_msg, W_self, b_gnn.reshape(1, H),
                    wcat, bcat)

    bidx = _pad_rows(bond_idx, NB_P).reshape(-1)
    aidx = _pad_rows(angle_idx, NA_P).reshape(-1)
    pidx = _pad_rows(proper_idx, NPR_P).reshape(-1)
    iidx = _pad_rows(improper_idx, NI_P).reshape(-1)
    bonds_f, angles_f, propers_f, impropers_f = _pools_sc(
        t.reshape(-1), bidx, aidx, pidx, iidx)

    atoms = t[14:16, :N].T
    bonds = bonds_f.reshape(NB_P, OUT)[:NB]
    angles = angles_f.reshape(NA_P, OUT)[:NA]
    propers = propers_f.reshape(NPR_P, OUT)[:NPR]
    impropers = impropers_f.reshape(NI_P, OUT)[:NI]
    return (atoms, bonds, angles, propers, impropers)


# speedup vs baseline: 4.1909x; 4.1909x over previous
import functools

import jax
import jax.numpy as jnp
from jax import lax
from jax.experimental import pallas as pl
from jax.experimental.pallas import tpu as pltpu
from jax.experimental.pallas import tpu_sc as plsc

N = 10000
E = 320000
D = 128
H = 128
OUT = 2
NB = 160000
NA = 150000
NPR = 100000
NI = 50000

NC = 2
NS = 16
L = 16

N_PAD = 10240
R_TC = 512

E_PER_SC = E // NC
E_PER_TILE = E_PER_SC // NS
SEG_CH = 88
SEG_FULL = E_PER_TILE // SEG_CH
SEG_PAIRS = (SEG_FULL - 1) // 2
SEG_TAIL = E_PER_TILE - SEG_FULL * SEG_CH
N_SH_PER_TILE = N_PAD // NS

NB_P = 160256
NA_P = 150016
NPR_P = 100352
NI_P = 50176
POOL_MAX_W = 5008

_mesh = plsc.VectorSubcoreMesh(core_axis_name="c", subcore_axis_name="s")


@functools.partial(
    pl.kernel,
    out_type=jax.ShapeDtypeStruct((NC, N_PAD, D), jnp.float32),
    mesh=_mesh,
    scratch_types=[
        pltpu.VMEM((E_PER_TILE,), jnp.int32),
        pltpu.VMEM((E_PER_TILE,), jnp.int32),
        pltpu.VMEM((SEG_CH, D), jnp.float32),
        pltpu.VMEM((SEG_CH, D), jnp.float32),
        pltpu.VMEM_SHARED((N_PAD, D), jnp.float32),
        pltpu.SemaphoreType.DMA,
        pltpu.SemaphoreType.DMA,
    ],
    compiler_params=pltpu.CompilerParams(needs_layout_passes=False),
)
def _segment_sum_sc(x_hbm, src_hbm, dst_hbm, zeros_hbm, out_hbm, srcs_v,
                    dsts_v, rows_a, rows_b, agg_sh, sem_a, sem_b):
    c = lax.axis_index("c")
    s = lax.axis_index("s")

    pltpu.sync_copy(zeros_hbm, agg_sh.at[pl.ds(s * N_SH_PER_TILE,
                                               N_SH_PER_TILE)])

    tile_base = c * E_PER_SC + s * E_PER_TILE
    pltpu.sync_copy(src_hbm.at[pl.ds(tile_base, E_PER_TILE)], srcs_v)
    pltpu.sync_copy(dst_hbm.at[pl.ds(tile_base, E_PER_TILE)], dsts_v)
    plsc.subcore_barrier()

    def _gather(off, rows, sem):
        return pltpu.async_copy(
            x_hbm.at[srcs_v.at[pl.ds(off, SEG_CH)]], rows, sem)

    def _wait(off, rows, sem):
        pltpu.make_async_copy(
            x_hbm.at[srcs_v.at[pl.ds(off, SEG_CH)]], rows, sem).wait()

    def _scatter_add(off, rows):
        pltpu.sync_copy(rows, agg_sh.at[dsts_v.at[pl.ds(off, SEG_CH)]],
                        add=True)

    _gather(0, rows_a, sem_a)

    def _edge_body(k, _):
        offa = 2 * k * SEG_CH
        offb = offa + SEG_CH
        offn = offa + 2 * SEG_CH
        _gather(offb, rows_b, sem_b)
        _wait(offa, rows_a, sem_a)
        _scatter_add(offa, rows_a)
        _gather(offn, rows_a, sem_a)
        _wait(offb, rows_b, sem_b)
        _scatter_add(offb, rows_b)
        return 0

    lax.fori_loop(0, SEG_PAIRS, _edge_body, 0)
    offl = (SEG_FULL - 1) * SEG_CH
    offt = SEG_FULL * SEG_CH
    tail_rows = rows_b.at[pl.ds(0, SEG_TAIL)]
    pltpu.async_copy(x_hbm.at[srcs_v.at[pl.ds(offt, SEG_TAIL)]], tail_rows,
                     sem_b)
    _wait(offl, rows_a, sem_a)
    _scatter_add(offl, rows_a)
    pltpu.make_async_copy(x_hbm.at[srcs_v.at[pl.ds(offt, SEG_TAIL)]],
                          tail_rows, sem_b).wait()
    pltpu.sync_copy(tail_rows, agg_sh.at[dsts_v.at[pl.ds(offt, SEG_TAIL)]],
                    add=True)
    plsc.subcore_barrier()

    pltpu.sync_copy(agg_sh.at[pl.ds(s * N_SH_PER_TILE, N_SH_PER_TILE)],
                    out_hbm.at[c, pl.ds(s * N_SH_PER_TILE, N_SH_PER_TILE)])


def _encoder_body(x_ref, agg_ref, wmsg_ref, wself_ref, bg_ref, wcat_ref,
                  bcat_ref, t_ref):
    agg = agg_ref[0] + agg_ref[1]
    pre = (
        lax.dot_general(agg, wmsg_ref[...], (((1,), (0,)), ((), ())),
                        precision=lax.Precision.HIGHEST)
        + lax.dot_general(x_ref[...], wself_ref[...], (((1,), (0,)), ((), ())),
                          precision=lax.Precision.HIGHEST)
        + bg_ref[...]
    )
    nr = jnp.maximum(pre, 0.0)
    t = lax.dot_general(wcat_ref[...], nr, (((0,), (1,)), ((), ())),
                        precision=lax.Precision.HIGHEST)
    t_ref[...] = t + bcat_ref[...]


_encoder_tc = pl.pallas_call(
    _encoder_body,
    grid=(N_PAD // R_TC,),
    in_specs=[
        pl.BlockSpec((R_TC, D), lambda i: (i, 0)),
        pl.BlockSpec((NC, R_TC, D), lambda i: (0, i, 0)),
        pl.BlockSpec((D, H), lambda i: (0, 0)),
        pl.BlockSpec((D, H), lambda i: (0, 0)),
        pl.BlockSpec((1, H), lambda i: (0, 0)),
        pl.BlockSpec((H, 16), lambda i: (0, 0)),
        pl.BlockSpec((16, 1), lambda i: (0, 0)),
    ],
    out_specs=pl.BlockSpec((16, R_TC), lambda i: (0, i)),
    out_shape=jax.ShapeDtypeStruct((16, N_PAD), jnp.float32),
)

_POOL_PHASES = (
    (NB_P, 2, (0, 1, 0, 1), (0, 0)),
    (NA_P, 3, (2, 3, 4, 5), (0, 2, 0)),
    (NPR_P, 4, (6, 7, 8, 9), (0, 2, 2, 0)),
    (NI_P, 4, (10, 11, 12, 13), (0, 2, 2, 2)),
)


@functools.partial(
    pl.kernel,
    out_type=tuple(
        jax.ShapeDtypeStruct((OUT * cnt,), jnp.float32)
        for (cnt, _, _, _) in _POOL_PHASES
    ),
    mesh=_mesh,
    scratch_types=[
        pltpu.VMEM((4 * N,), jnp.float32),
        pltpu.VMEM((4 * POOL_MAX_W,), jnp.int32),
        pltpu.VMEM((OUT * POOL_MAX_W,), jnp.float32),
    ],
    compiler_params=pltpu.CompilerParams(needs_layout_passes=False),
)
def _pools_sc(t_hbm, bidx_hbm, aidx_hbm, pidx_hbm, iidx_hbm,
              bonds_hbm, angles_hbm, propers_hbm, impropers_hbm,
              tab_v, idx_v, out_v):
    c = lax.axis_index("c")
    s = lax.axis_index("s")
    wid = s * NC + c

    idx_refs = (bidx_hbm, aidx_hbm, pidx_hbm, iidx_hbm)
    out_refs = (bonds_hbm, angles_hbm, propers_hbm, impropers_hbm)

    for phase, (cnt, k, trows, tbase) in enumerate(_POOL_PHASES):
        w = cnt // (NC * NS)
        base = wid * w
        for j, r in enumerate(trows):
            pltpu.sync_copy(t_hbm.at[pl.ds(r * N_PAD, N)],
                            tab_v.at[pl.ds(j * N, N)])
        for j in range(k):
            pltpu.sync_copy(idx_refs[phase].at[pl.ds(j * cnt + base, w)],
                            idx_v.at[pl.ds(j * POOL_MAX_W, w)])

        def _group_body(g, _, k=k, tbase=tbase):
            off = g * L
            ivecs = [idx_v[pl.ds(j * POOL_MAX_W + off, L)] for j in range(k)]
            for col in range(OUT):
                acc = plsc.load_gather(
                    tab_v, [ivecs[0] + (tbase[0] + col) * N])
                for j in range(1, k):
                    acc = acc + plsc.load_gather(
                        tab_v, [ivecs[j] + (tbase[j] + col) * N])
                out_v[pl.ds(col * POOL_MAX_W + off, L)] = acc
            return 0

        lax.fori_loop(0, w // L, _group_body, 0)
        for col in range(OUT):
            pltpu.sync_copy(out_v.at[pl.ds(col * POOL_MAX_W, w)],
                            out_refs[phase].at[pl.ds(col * cnt + base, w)])


def _pad_rows(a, rows):
    return jnp.pad(a, ((0, rows - a.shape[0]), (0, 0)))


def kernel(x, edge_index, bond_idx, angle_idx, proper_idx, improper_idx,
           W_msg, W_self, b_gnn, W_atoms, b_atoms, W_bonds, b_bonds,
           W_angles, b_angles, W_propers, b_propers, W_impropers, b_impropers):
    f32 = jnp.float32

    wb = W_bonds[:H] + W_bonds[H:]
    wa02 = W_angles[:H] + W_angles[2 * H:]
    wa1 = 2.0 * W_angles[H:2 * H]
    wp03 = W_propers[:H] + W_propers[3 * H:]
    wp12 = W_propers[H:2 * H] + W_propers[2 * H:3 * H]
    wi0 = 3.0 * W_impropers[:H]
    wi123 = (W_impropers[H:2 * H] + W_impropers[2 * H:3 * H]
             + W_impropers[3 * H:])
    wcat = jnp.concatenate(
        [wb, wa02, wa1, wp03, wp12, wi0, wi123, W_atoms], axis=1)
    z2 = jnp.zeros((OUT,), f32)
    bcat = jnp.concatenate(
        [b_bonds, z2, 2.0 * b_angles, b_propers, z2, 3.0 * b_impropers, z2,
         b_atoms]).reshape(16, 1)

    zeros_blk = jnp.zeros((N_SH_PER_TILE, D), f32)
    agg_pair = _segment_sum_sc(x, edge_index[0], edge_index[1], zeros_blk)

    x_pad = _pad_rows(x, N_PAD)
    t = _encoder_tc(x_pad, agg_pair, W_msg, W_self, b_gnn.reshape(1, H),
                    wcat, bcat)

    bidx = _pad_rows(bond_idx, NB_P).T.reshape(-1)
    aidx = _pad_rows(angle_idx, NA_P).T.reshape(-1)
    pidx = _pad_rows(proper_idx, NPR_P).T.reshape(-1)
    iidx = _pad_rows(improper_idx, NI_P).T.reshape(-1)
    bonds_f, angles_f, propers_f, impropers_f = _pools_sc(
        t.reshape(-1), bidx, aidx, pidx, iidx)

    atoms = t[14:16, :N].T
    bonds = bonds_f.reshape(OUT, NB_P)[:, :NB].T
    angles = angles_f.reshape(OUT, NA_P)[:, :NA].T
    propers = propers_f.reshape(OUT, NPR_P)[:, :NPR].T
    impropers = impropers_f.reshape(OUT, NI_P)[:, :NI].T
    return (atoms, bonds, angles, propers, impropers)


# speedup vs baseline: 4.5315x; 1.0813x over previous
import functools

import jax
import jax.numpy as jnp
from jax import lax
from jax.experimental import pallas as pl
from jax.experimental.pallas import tpu as pltpu
from jax.experimental.pallas import tpu_sc as plsc

N = 10000
E = 320000
D = 128
H = 128
OUT = 2
NB = 160000
NA = 150000
NPR = 100000
NI = 50000

NC = 2
NS = 16
L = 16

N_PAD = 10240
R_TC = 512

E_PER_SC = E // NC
E_PER_TILE = E_PER_SC // NS
SEG_CH = 88
SEG_FULL = E_PER_TILE // SEG_CH
SEG_PAIRS = (SEG_FULL - 1) // 2
SEG_TAIL = E_PER_TILE - SEG_FULL * SEG_CH
N_SH_PER_TILE = N_PAD // NS

NB_P = 160256
NA_P = 150016
NPR_P = 100352
NI_P = 50176
POOL_MAX_W = 5008

_mesh = plsc.VectorSubcoreMesh(core_axis_name="c", subcore_axis_name="s")


@functools.partial(
    pl.kernel,
    out_type=jax.ShapeDtypeStruct((NC, N_PAD, D), jnp.float32),
    mesh=_mesh,
    scratch_types=[
        pltpu.VMEM((E_PER_TILE,), jnp.int32),
        pltpu.VMEM((E_PER_TILE,), jnp.int32),
        pltpu.VMEM((SEG_CH, D), jnp.float32),
        pltpu.VMEM((SEG_CH, D), jnp.float32),
        pltpu.VMEM_SHARED((N_PAD, D), jnp.float32),
        pltpu.SemaphoreType.DMA,
        pltpu.SemaphoreType.DMA,
    ],
    compiler_params=pltpu.CompilerParams(needs_layout_passes=False),
)
def _segment_sum_sc(x_hbm, src_hbm, dst_hbm, zeros_hbm, out_hbm, srcs_v,
                    dsts_v, rows_a, rows_b, agg_sh, sem_a, sem_b):
    c = lax.axis_index("c")
    s = lax.axis_index("s")

    pltpu.sync_copy(zeros_hbm, agg_sh.at[pl.ds(s * N_SH_PER_TILE,
                                               N_SH_PER_TILE)])

    tile_base = c * E_PER_SC + s * E_PER_TILE
    pltpu.sync_copy(src_hbm.at[pl.ds(tile_base, E_PER_TILE)], srcs_v)
    pltpu.sync_copy(dst_hbm.at[pl.ds(tile_base, E_PER_TILE)], dsts_v)
    plsc.subcore_barrier()

    def _gather(off, rows, sem):
        return pltpu.async_copy(
            x_hbm.at[srcs_v.at[pl.ds(off, SEG_CH)]], rows, sem)

    def _wait(off, rows, sem):
        pltpu.make_async_copy(
            x_hbm.at[srcs_v.at[pl.ds(off, SEG_CH)]], rows, sem).wait()

    def _scatter_add(off, rows):
        pltpu.sync_copy(rows, agg_sh.at[dsts_v.at[pl.ds(off, SEG_CH)]],
                        add=True)

    _gather(0, rows_a, sem_a)

    def _edge_body(k, _):
        offa = 2 * k * SEG_CH
        offb = offa + SEG_CH
        offn = offa + 2 * SEG_CH
        _gather(offb, rows_b, sem_b)
        _wait(offa, rows_a, sem_a)
        _scatter_add(offa, rows_a)
        _gather(offn, rows_a, sem_a)
        _wait(offb, rows_b, sem_b)
        _scatter_add(offb, rows_b)
        return 0

    lax.fori_loop(0, SEG_PAIRS, _edge_body, 0)
    offl = (SEG_FULL - 1) * SEG_CH
    offt = SEG_FULL * SEG_CH
    tail_rows = rows_b.at[pl.ds(0, SEG_TAIL)]
    pltpu.async_copy(x_hbm.at[srcs_v.at[pl.ds(offt, SEG_TAIL)]], tail_rows,
                     sem_b)
    _wait(offl, rows_a, sem_a)
    _scatter_add(offl, rows_a)
    pltpu.make_async_copy(x_hbm.at[srcs_v.at[pl.ds(offt, SEG_TAIL)]],
                          tail_rows, sem_b).wait()
    pltpu.sync_copy(tail_rows, agg_sh.at[dsts_v.at[pl.ds(offt, SEG_TAIL)]],
                    add=True)
    plsc.subcore_barrier()

    pltpu.sync_copy(agg_sh.at[pl.ds(s * N_SH_PER_TILE, N_SH_PER_TILE)],
                    out_hbm.at[c, pl.ds(s * N_SH_PER_TILE, N_SH_PER_TILE)])


def _encoder_body(x_ref, agg_ref, wmsg_ref, wself_ref, bg_ref, wcat_ref,
                  bcat_ref, t_ref):
    agg = agg_ref[0] + agg_ref[1]
    pre = (
        lax.dot_general(agg, wmsg_ref[...], (((1,), (0,)), ((), ())),
                        precision=lax.Precision.HIGHEST)
        + lax.dot_general(x_ref[...], wself_ref[...], (((1,), (0,)), ((), ())),
                          precision=lax.Precision.HIGHEST)
        + bg_ref[...]
    )
    nr = jnp.maximum(pre, 0.0)
    t = lax.dot_general(wcat_ref[...], nr, (((0,), (1,)), ((), ())),
                        precision=lax.Precision.HIGHEST)
    t_ref[...] = t + bcat_ref[...]


_encoder_tc = pl.pallas_call(
    _encoder_body,
    grid=(N_PAD // R_TC,),
    in_specs=[
        pl.BlockSpec((R_TC, D), lambda i: (i, 0)),
        pl.BlockSpec((NC, R_TC, D), lambda i: (0, i, 0)),
        pl.BlockSpec((D, H), lambda i: (0, 0)),
        pl.BlockSpec((D, H), lambda i: (0, 0)),
        pl.BlockSpec((1, H), lambda i: (0, 0)),
        pl.BlockSpec((H, 16), lambda i: (0, 0)),
        pl.BlockSpec((16, 1), lambda i: (0, 0)),
    ],
    out_specs=pl.BlockSpec((16, R_TC), lambda i: (0, i)),
    out_shape=jax.ShapeDtypeStruct((16, N_PAD), jnp.float32),
)

_POOL_PHASES = (
    (NB_P, 2, (0, 1), (0, 0), 2, 10016),
    (NA_P, 3, (2, 3, 4, 5), (0, 2, 0), 2, 9376),
    (NPR_P, 4, (6, 7, 8, 9), (0, 2, 2, 0), 1, 12544),
    (NI_P, 4, (10, 11, 12, 13), (0, 2, 2, 2), 1, 6272),
)
POOL_TPP = (NC * NS) // len(_POOL_PHASES)
POOL_IDX_MAX = 50176
POOL_OUT_MAX = 25088


@functools.partial(
    pl.kernel,
    out_type=tuple(
        jax.ShapeDtypeStruct((OUT * cnt,), jnp.float32)
        for (cnt, _, _, _, _, _) in _POOL_PHASES
    ),
    mesh=_mesh,
    scratch_types=[
        pltpu.VMEM((4 * N,), jnp.float32),
        pltpu.VMEM((POOL_IDX_MAX,), jnp.int32),
        pltpu.VMEM((POOL_OUT_MAX,), jnp.float32),
    ],
    compiler_params=pltpu.CompilerParams(needs_layout_passes=False),
)
def _pools_sc(t_hbm, bidx_hbm, aidx_hbm, pidx_hbm, iidx_hbm,
              bonds_hbm, angles_hbm, propers_hbm, impropers_hbm,
              tab_v, idx_v, out_v):
    c = lax.axis_index("c")
    s = lax.axis_index("s")
    wid = s * NC + c

    idx_refs = (bidx_hbm, aidx_hbm, pidx_hbm, iidx_hbm)
    out_refs = (bonds_hbm, angles_hbm, propers_hbm, impropers_hbm)

    for phase, (cnt, k, trows, tbase, nch, csz) in enumerate(_POOL_PHASES):
        w = cnt // POOL_TPP

        @pl.when((wid >= phase * POOL_TPP) & (wid < (phase + 1) * POOL_TPP))
        def _phase_body(phase=phase, cnt=cnt, k=k, trows=trows, tbase=tbase,
                        nch=nch, csz=csz, w=w):
            q = wid - phase * POOL_TPP
            for j, r in enumerate(trows):
                pltpu.sync_copy(t_hbm.at[pl.ds(r * N_PAD, N)],
                                tab_v.at[pl.ds(j * N, N)])
            for ci in range(nch):
                base = q * w + ci * csz
                for j in range(k):
                    pltpu.sync_copy(
                        idx_refs[phase].at[pl.ds(j * cnt + base, csz)],
                        idx_v.at[pl.ds(j * csz, csz)])

                def _group_body(g, _, k=k, tbase=tbase, csz=csz):
                    off = g * L
                    ivecs = [idx_v[pl.ds(j * csz + off, L)] for j in range(k)]
                    for col in range(OUT):
                        acc = plsc.load_gather(
                            tab_v, [ivecs[0] + (tbase[0] + col) * N])
                        for j in range(1, k):
                            acc = acc + plsc.load_gather(
                                tab_v, [ivecs[j] + (tbase[j] + col) * N])
                        out_v[pl.ds(col * csz + off, L)] = acc
                    return 0

                lax.fori_loop(0, csz // L, _group_body, 0)
                for col in range(OUT):
                    pltpu.sync_copy(
                        out_v.at[pl.ds(col * csz, csz)],
                        out_refs[phase].at[pl.ds(col * cnt + base, csz)])


def _pad_rows(a, rows):
    return jnp.pad(a, ((0, rows - a.shape[0]), (0, 0)))


def kernel(x, edge_index, bond_idx, angle_idx, proper_idx, improper_idx,
           W_msg, W_self, b_gnn, W_atoms, b_atoms, W_bonds, b_bonds,
           W_angles, b_angles, W_propers, b_propers, W_impropers, b_impropers):
    f32 = jnp.float32

    wb = W_bonds[:H] + W_bonds[H:]
    wa02 = W_angles[:H] + W_angles[2 * H:]
    wa1 = 2.0 * W_angles[H:2 * H]
    wp03 = W_propers[:H] + W_propers[3 * H:]
    wp12 = W_propers[H:2 * H] + W_propers[2 * H:3 * H]
    wi0 = 3.0 * W_impropers[:H]
    wi123 = (W_impropers[H:2 * H] + W_impropers[2 * H:3 * H]
             + W_impropers[3 * H:])
    wcat = jnp.concatenate(
        [wb, wa02, wa1, wp03, wp12, wi0, wi123, W_atoms], axis=1)
    z2 = jnp.zeros((OUT,), f32)
    bcat = jnp.concatenate(
        [b_bonds, z2, 2.0 * b_angles, b_propers, z2, 3.0 * b_impropers, z2,
         b_atoms]).reshape(16, 1)

    zeros_blk = jnp.zeros((N_SH_PER_TILE, D), f32)
    agg_pair = _segment_sum_sc(x, edge_index[0], edge_index[1], zeros_blk)

    x_pad = _pad_rows(x, N_PAD)
    t = _encoder_tc(x_pad, agg_pair, W_msg, W_self, b_gnn.reshape(1, H),
                    wcat, bcat)

    bidx = _pad_rows(bond_idx, NB_P).T.reshape(-1)
    aidx = _pad_rows(angle_idx, NA_P).T.reshape(-1)
    pidx = _pad_rows(proper_idx, NPR_P).T.reshape(-1)
    iidx = _pad_rows(improper_idx, NI_P).T.reshape(-1)
    bonds_f, angles_f, propers_f, impropers_f = _pools_sc(
        t.reshape(-1), bidx, aidx, pidx, iidx)

    atoms = t[14:16, :N].T
    bonds = bonds_f.reshape(OUT, NB_P)[:, :NB].T
    angles = angles_f.reshape(OUT, NA_P)[:, :NA].T
    propers = propers_f.reshape(OUT, NPR_P)[:, :NPR].T
    impropers = impropers_f.reshape(OUT, NI_P)[:, :NI].T
    return (atoms, bonds, angles, propers, impropers)


# speedup vs baseline: 4.5906x; 1.0131x over previous
import functools

import jax
import jax.numpy as jnp
from jax import lax
from jax.experimental import pallas as pl
from jax.experimental.pallas import tpu as pltpu
from jax.experimental.pallas import tpu_sc as plsc

N = 10000
E = 320000
D = 128
H = 128
OUT = 2
NB = 160000
NA = 150000
NPR = 100000
NI = 50000

NC = 2
NS = 16
L = 16

N_PAD = 10240
R_TC = 1024

E_PER_SC = E // NC
E_PER_TILE = E_PER_SC // NS
SEG_CH = 88
SEG_FULL = E_PER_TILE // SEG_CH
SEG_PAIRS = (SEG_FULL - 1) // 2
SEG_TAIL = E_PER_TILE - SEG_FULL * SEG_CH
N_SH_PER_TILE = N_PAD // NS

NB_P = 160256
NA_P = 150016
NPR_P = 100352
NI_P = 50176
POOL_MAX_W = 5008

_mesh = plsc.VectorSubcoreMesh(core_axis_name="c", subcore_axis_name="s")


@functools.partial(
    pl.kernel,
    out_type=jax.ShapeDtypeStruct((NC, N_PAD, D), jnp.float32),
    mesh=_mesh,
    scratch_types=[
        pltpu.VMEM((E_PER_TILE,), jnp.int32),
        pltpu.VMEM((E_PER_TILE,), jnp.int32),
        pltpu.VMEM((SEG_CH, D), jnp.float32),
        pltpu.VMEM((SEG_CH, D), jnp.float32),
        pltpu.VMEM_SHARED((N_PAD, D), jnp.float32),
        pltpu.SemaphoreType.DMA,
        pltpu.SemaphoreType.DMA,
    ],
    compiler_params=pltpu.CompilerParams(needs_layout_passes=False),
)
def _segment_sum_sc(x_hbm, src_hbm, dst_hbm, zeros_hbm, out_hbm, srcs_v,
                    dsts_v, rows_a, rows_b, agg_sh, sem_a, sem_b):
    c = lax.axis_index("c")
    s = lax.axis_index("s")

    pltpu.sync_copy(zeros_hbm, agg_sh.at[pl.ds(s * N_SH_PER_TILE,
                                               N_SH_PER_TILE)])

    tile_base = c * E_PER_SC + s * E_PER_TILE
    pltpu.sync_copy(src_hbm.at[pl.ds(tile_base, E_PER_TILE)], srcs_v)
    pltpu.sync_copy(dst_hbm.at[pl.ds(tile_base, E_PER_TILE)], dsts_v)
    plsc.subcore_barrier()

    def _gather(off, rows, sem):
        return pltpu.async_copy(
            x_hbm.at[srcs_v.at[pl.ds(off, SEG_CH)]], rows, sem)

    def _wait(off, rows, sem):
        pltpu.make_async_copy(
            x_hbm.at[srcs_v.at[pl.ds(off, SEG_CH)]], rows, sem).wait()

    def _scatter_add(off, rows):
        pltpu.sync_copy(rows, agg_sh.at[dsts_v.at[pl.ds(off, SEG_CH)]],
                        add=True)

    _gather(0, rows_a, sem_a)

    def _edge_body(k, _):
        offa = 2 * k * SEG_CH
        offb = offa + SEG_CH
        offn = offa + 2 * SEG_CH
        _gather(offb, rows_b, sem_b)
        _wait(offa, rows_a, sem_a)
        _scatter_add(offa, rows_a)
        _gather(offn, rows_a, sem_a)
        _wait(offb, rows_b, sem_b)
        _scatter_add(offb, rows_b)
        return 0

    lax.fori_loop(0, SEG_PAIRS, _edge_body, 0)
    offl = (SEG_FULL - 1) * SEG_CH
    offt = SEG_FULL * SEG_CH
    tail_rows = rows_b.at[pl.ds(0, SEG_TAIL)]
    pltpu.async_copy(x_hbm.at[srcs_v.at[pl.ds(offt, SEG_TAIL)]], tail_rows,
                     sem_b)
    _wait(offl, rows_a, sem_a)
    _scatter_add(offl, rows_a)
    pltpu.make_async_copy(x_hbm.at[srcs_v.at[pl.ds(offt, SEG_TAIL)]],
                          tail_rows, sem_b).wait()
    pltpu.sync_copy(tail_rows, agg_sh.at[dsts_v.at[pl.ds(offt, SEG_TAIL)]],
                    add=True)
    plsc.subcore_barrier()

    pltpu.sync_copy(agg_sh.at[pl.ds(s * N_SH_PER_TILE, N_SH_PER_TILE)],
                    out_hbm.at[c, pl.ds(s * N_SH_PER_TILE, N_SH_PER_TILE)])


def _encoder_body(x_ref, agg_ref, wmsg_ref, wself_ref, bg_ref, wcat_ref,
                  bcat_ref, t_ref):
    agg = agg_ref[0] + agg_ref[1]
    pre = (
        lax.dot_general(agg, wmsg_ref[...], (((1,), (0,)), ((), ())),
                        precision=lax.Precision.HIGHEST)
        + lax.dot_general(x_ref[...], wself_ref[...], (((1,), (0,)), ((), ())),
                          precision=lax.Precision.HIGHEST)
        + bg_ref[...]
    )
    nr = jnp.maximum(pre, 0.0)
    t = lax.dot_general(wcat_ref[...], nr, (((0,), (1,)), ((), ())),
                        precision=lax.Precision.HIGHEST)
    t_ref[...] = t + bcat_ref[...]


_encoder_tc = pl.pallas_call(
    _encoder_body,
    grid=(N_PAD // R_TC,),
    in_specs=[
        pl.BlockSpec((R_TC, D), lambda i: (i, 0)),
        pl.BlockSpec((NC, R_TC, D), lambda i: (0, i, 0)),
        pl.BlockSpec((D, H), lambda i: (0, 0)),
        pl.BlockSpec((D, H), lambda i: (0, 0)),
        pl.BlockSpec((1, H), lambda i: (0, 0)),
        pl.BlockSpec((H, 16), lambda i: (0, 0)),
        pl.BlockSpec((16, 1), lambda i: (0, 0)),
    ],
    out_specs=pl.BlockSpec((16, R_TC), lambda i: (0, i)),
    out_shape=jax.ShapeDtypeStruct((16, N_PAD), jnp.float32),
)

_POOL_PHASES = (
    (NB_P, 2, (0, 1), (0, 0), 2, 10016),
    (NA_P, 3, (2, 3, 4, 5), (0, 2, 0), 2, 9376),
    (NPR_P, 4, (6, 7, 8, 9), (0, 2, 2, 0), 1, 12544),
    (NI_P, 4, (10, 11, 12, 13), (0, 2, 2, 2), 1, 6272),
)
POOL_TPP = (NC * NS) // len(_POOL_PHASES)
POOL_IDX_MAX = 50176
POOL_OUT_MAX = 25088


@functools.partial(
    pl.kernel,
    out_type=tuple(
        jax.ShapeDtypeStruct((OUT * cnt,), jnp.float32)
        for (cnt, _, _, _, _, _) in _POOL_PHASES
    ),
    mesh=_mesh,
    scratch_types=[
        pltpu.VMEM((4 * N,), jnp.float32),
        pltpu.VMEM((POOL_IDX_MAX,), jnp.int32),
        pltpu.VMEM((POOL_OUT_MAX,), jnp.float32),
    ],
    compiler_params=pltpu.CompilerParams(needs_layout_passes=False),
)
def _pools_sc(t_hbm, bidx_hbm, aidx_hbm, pidx_hbm, iidx_hbm,
              bonds_hbm, angles_hbm, propers_hbm, impropers_hbm,
              tab_v, idx_v, out_v):
    c = lax.axis_index("c")
    s = lax.axis_index("s")
    wid = s * NC + c

    idx_refs = (bidx_hbm, aidx_hbm, pidx_hbm, iidx_hbm)
    out_refs = (bonds_hbm, angles_hbm, propers_hbm, impropers_hbm)

    for phase, (cnt, k, trows, tbase, nch, csz) in enumerate(_POOL_PHASES):
        w = cnt // POOL_TPP

        @pl.when((wid >= phase * POOL_TPP) & (wid < (phase + 1) * POOL_TPP))
        def _phase_body(phase=phase, cnt=cnt, k=k, trows=trows, tbase=tbase,
                        nch=nch, csz=csz, w=w):
            q = wid - phase * POOL_TPP
            for j, r in enumerate(trows):
                pltpu.sync_copy(t_hbm.at[pl.ds(r * N_PAD, N)],
                                tab_v.at[pl.ds(j * N, N)])
            for ci in range(nch):
                base = q * w + ci * csz
                for j in range(k):
                    pltpu.sync_copy(
                        idx_refs[phase].at[pl.ds(j * cnt + base, csz)],
                        idx_v.at[pl.ds(j * csz, csz)])

                def _group_body(g, _, k=k, tbase=tbase, csz=csz):
                    off = g * L
                    ivecs = [idx_v[pl.ds(j * csz + off, L)] for j in range(k)]
                    for col in range(OUT):
                        acc = plsc.load_gather(
                            tab_v, [ivecs[0] + (tbase[0] + col) * N])
                        for j in range(1, k):
                            acc = acc + plsc.load_gather(
                                tab_v, [ivecs[j] + (tbase[j] + col) * N])
                        out_v[pl.ds(col * csz + off, L)] = acc
                    return 0

                lax.fori_loop(0, csz // L, _group_body, 0)
                for col in range(OUT):
                    pltpu.sync_copy(
                        out_v.at[pl.ds(col * csz, csz)],
                        out_refs[phase].at[pl.ds(col * cnt + base, csz)])


def _pad_rows(a, rows):
    return jnp.pad(a, ((0, rows - a.shape[0]), (0, 0)))


def kernel(x, edge_index, bond_idx, angle_idx, proper_idx, improper_idx,
           W_msg, W_self, b_gnn, W_atoms, b_atoms, W_bonds, b_bonds,
           W_angles, b_angles, W_propers, b_propers, W_impropers, b_impropers):
    f32 = jnp.float32

    wb = W_bonds[:H] + W_bonds[H:]
    wa02 = W_angles[:H] + W_angles[2 * H:]
    wa1 = 2.0 * W_angles[H:2 * H]
    wp03 = W_propers[:H] + W_propers[3 * H:]
    wp12 = W_propers[H:2 * H] + W_propers[2 * H:3 * H]
    wi0 = 3.0 * W_impropers[:H]
    wi123 = (W_impropers[H:2 * H] + W_impropers[2 * H:3 * H]
             + W_impropers[3 * H:])
    wcat = jnp.concatenate(
        [wb, wa02, wa1, wp03, wp12, wi0, wi123, W_atoms], axis=1)
    z2 = jnp.zeros((OUT,), f32)
    bcat = jnp.concatenate(
        [b_bonds, z2, 2.0 * b_angles, b_propers, z2, 3.0 * b_impropers, z2,
         b_atoms]).reshape(16, 1)

    zeros_blk = jnp.zeros((N_SH_PER_TILE, D), f32)
    agg_pair = _segment_sum_sc(x, edge_index[0], edge_index[1], zeros_blk)

    x_pad = _pad_rows(x, N_PAD)
    t = _encoder_tc(x_pad, agg_pair, W_msg, W_self, b_gnn.reshape(1, H),
                    wcat, bcat)

    bidx = _pad_rows(bond_idx, NB_P).T.reshape(-1)
    aidx = _pad_rows(angle_idx, NA_P).T.reshape(-1)
    pidx = _pad_rows(proper_idx, NPR_P).T.reshape(-1)
    iidx = _pad_rows(improper_idx, NI_P).T.reshape(-1)
    bonds_f, angles_f, propers_f, impropers_f = _pools_sc(
        t.reshape(-1), bidx, aidx, pidx, iidx)

    atoms = t[14:16, :N].T
    bonds = bonds_f.reshape(OUT, NB_P)[:, :NB].T
    angles = angles_f.reshape(OUT, NA_P)[:, :NA].T
    propers = propers_f.reshape(OUT, NPR_P)[:, :NPR].T
    impropers = impropers_f.reshape(OUT, NI_P)[:, :NI].T
    return (atoms, bonds, angles, propers, impropers)
